# baseline (speedup 1.0000x reference)
import functools
import math

import numpy as np

import jax
import jax.numpy as jnp
from jax import lax
from jax.experimental import pallas as pl
from jax.experimental.pallas import tpu as pltpu
from jax.experimental.pallas import tpu_sc as plsc

STD2 = 0.1
VAR2 = STD2 * STD2
LOG_VAR2 = float(math.log(VAR2))
LN_EPS = 1e-5

N_Q = 8
N_R = 4096
D_IN = 384
D_CTX = 128
HID = 64
RB = 2048
NB = N_R // RB

_ROT1 = (13, 15, 26, 6)
_ROT2 = (17, 29, 16, 24)


def _np_rotl(x, r):
    return ((x << np.uint32(r)) | (x >> np.uint32(32 - r))).astype(np.uint32)


def _np_tf_pair(key, x0, x1):
    ks0, ks1 = np.uint32(key[0]), np.uint32(key[1])
    ks2 = np.uint32(ks0 ^ ks1 ^ np.uint32(0x1BD11BDA))
    x0 = (x0 + ks0).astype(np.uint32)
    x1 = (x1 + ks1).astype(np.uint32)
    for rots, a0, a1, c in [(_ROT1, ks1, ks2, 1), (_ROT2, ks2, ks0, 2),
                            (_ROT1, ks0, ks1, 3), (_ROT2, ks1, ks2, 4),
                            (_ROT1, ks2, ks0, 5)]:
        for r in rots:
            x0 = (x0 + x1).astype(np.uint32)
            x1 = _np_rotl(x1, r)
            x1 = (x1 ^ x0).astype(np.uint32)
        x0 = (x0 + a0).astype(np.uint32)
        x1 = (x1 + a1 + np.uint32(c)).astype(np.uint32)
    return x0, x1


def _np_fold_in(key, i):
    o0, o1 = _np_tf_pair(key, np.array([0], np.uint32),
                         np.array([i], np.uint32))
    return np.array([o0[0], o1[0]], np.uint32)


_EPS_KEYS = np.stack([_np_fold_in(np.array([0, 1], np.uint32), i)
                      for i in range(N_Q)], axis=1)

def _np_rnd(i):
    kf = _np_fold_in(_np_fold_in(np.array([0, 2], np.uint32), i), 0)
    o0, o1 = _np_tf_pair(kf, np.zeros(1, np.uint32), np.zeros(1, np.uint32))
    fb = (((o0 ^ o1) >> np.uint32(9)) | np.uint32(0x3F800000)).view(np.float32)
    return float(np.maximum(np.float32(0.0),
                            (fb - np.float32(1.0)).astype(np.float32))[0])

_RND = np.array([[_np_rnd(i) for i in range(N_Q)]], np.float32)

_LO = float(np.nextafter(np.float32(-1), np.float32(0)))
_HL = float(np.float32(1.0) - np.float32(_LO))
_SQRT2 = float(np.float32(np.sqrt(np.float64(2.0))))
_P1C = [3.43273939e-07, -3.5233877e-06, -4.39150654e-06, 0.00021858087,
        -0.00125372503, -0.00417768164, 0.246640727, 1.50140941]
_P2C = [0.000100950558, 0.00134934322, -0.00367342844, 0.00573950773,
        -0.0076224613, 0.00943887047, 1.00167406, 2.83297682]


def _shr(x, s):
    n = x.shape[-1]
    return jnp.concatenate([x[:, n - s:], x[:, :n - s]], axis=1)


def _bk_cumsum(x, iota):
    for d in range(12):
        s = 1 << d
        m = (iota & (2 * s - 1)) == (2 * s - 1)
        x = jnp.where(m, x + _shr(x, s), x)
    for d in range(10, -1, -1):
        s = 1 << d
        m = ((iota & (2 * s - 1)) == (s - 1)) & (iota >= 3 * s - 1)
        x = jnp.where(m, x + _shr(x, s), x)
    return x


def _ln(x):
    mu = jnp.mean(x, axis=-1, keepdims=True)
    var = jnp.mean((x - mu) * (x - mu), axis=-1, keepdims=True)
    return (x - mu) / jnp.sqrt(var + LN_EPS)


def _nrm(x):
    n = jnp.sqrt(jnp.sum(x * x, axis=1, keepdims=True))
    return x / jnp.maximum(n, 1e-12)


def _tf_bits(ks0, ks1, x1):
    ks2 = ks0 ^ ks1 ^ jnp.uint32(0x1BD11BDA)
    x0 = jnp.zeros_like(x1) + ks0
    x1 = x1 + ks1
    for rots, a0, a1, c in [(_ROT1, ks1, ks2, 1), (_ROT2, ks2, ks0, 2),
                            (_ROT1, ks0, ks1, 3), (_ROT2, ks1, ks2, 4),
                            (_ROT1, ks2, ks0, 5)]:
        for r in rots:
            x0 = x0 + x1
            x1 = (x1 << jnp.uint32(r)) | (x1 >> jnp.uint32(32 - r))
            x1 = x1 ^ x0
        x0 = x0 + a0
        x1 = x1 + a1 + jnp.uint32(c)
    return x0 ^ x1


def _eps_block(ks0, ks1, j):
    half = RB // 2
    r_i = lax.broadcasted_iota(jnp.uint32, (half, 128), 0)
    c_i = lax.broadcasted_iota(jnp.uint32, (half, 128), 1)
    e = r_i * jnp.uint32(128) + c_i + (j * (RB * HID)).astype(jnp.uint32)
    bits = _tf_bits(ks0, ks1, e)
    fb = (bits >> jnp.uint32(9)) | jnp.uint32(0x3F800000)
    f = lax.bitcast_convert_type(fb, jnp.float32)
    u = (f - 1.0) * _HL + _LO
    u = jnp.maximum(_LO, u)
    w = -jnp.log1p(-(u * u))
    w1 = w - 2.5
    p1 = jnp.float32(2.81022636e-08)
    for c in _P1C:
        p1 = c + p1 * w1
    w2 = jnp.sqrt(w) - 3.0
    p2 = jnp.float32(-0.000200214257)
    for c in _P2C:
        p2 = c + p2 * w2
    p = jnp.where(w < 5.0, p1, p2)
    v = _SQRT2 * (p * u)
    vl = v[:, :HID].reshape(half, 1, HID)
    vr = v[:, HID:].reshape(half, 1, HID)
    return jnp.concatenate([vl, vr], axis=1).reshape(RB, HID)


def _body(roles_ref, ctx_ref, agent_ref, init_ref,
          w1_ref, b1_ref, w21_ref, b21_ref, w22_ref, b22_ref,
          w3_ref, b3_ref, w4_ref, b4_ref, cw_ref, cb_ref,
          key_ref,
          cs_ref, ls_ref, sum_ref, loss_ref,
          ctx_scr, log_scr, acc_ref):
    i = pl.program_id(0)
    j = pl.program_id(1)

    @pl.when(j == 0)
    def _prologue():
        init = init_ref[...]
        hn = _ln(init + init)
        act = agent_ref[0, i] > 0
        sum_ref[pl.ds(i, 1), :] = jnp.where(act, hn, init)
        ce = (ctx_ref[0] @ cw_ref[:D_CTX, :]
              + hn @ cw_ref[D_CTX:, :] + cb_ref[...])
        ctx_scr[...] = _nrm(ce)
        acc_ref[0, 0] = 0.0
        acc_ref[0, 1] = 0.0

        @pl.when(i == 0)
        def _():
            acc_ref[0, 2] = 0.0

    roles = roles_ref[0]
    h = jnp.maximum(roles @ w1_ref[...] + b1_ref[...], 0.0)
    mu = h @ w21_ref[...] + b21_ref[...]
    lv = h @ w22_ref[...] + b22_ref[...]
    ex = jnp.exp(0.5 * lv)
    eps = _eps_block(key_ref[0, i], key_ref[1, i], j)
    z = mu + eps * (ex * STD2)
    h2 = jnp.maximum(z @ w3_ref[...] + b3_ref[...], 0.0)
    xh = h2 @ w4_ref[...] + b4_ref[...]
    d = xh - roles
    acc_ref[0, 0] += jnp.sum(d * d)
    kterm = 1.0 - LOG_VAR2 + lv - (mu * mu + ex * ex) / VAR2
    acc_ref[0, 1] += jnp.sum(kterm)

    re = _nrm(z)
    lgt = lax.dot_general(ctx_scr[...], re,
                          (((1,), (1,)), ((), ())),
                          preferred_element_type=jnp.float32)
    log_scr[0:1, pl.ds(j * RB, RB)] = lgt

    @pl.when(j == NB - 1)
    def _sample():
        lg = log_scr[...]
        e = jnp.exp(lg - jnp.max(lg))
        sc = e / jnp.sum(e)
        iota = lax.broadcasted_iota(jnp.int32, (1, N_R), 1)
        cs_ref[0] = _bk_cumsum(sc, iota)
        ls_ref[0] = jnp.log(sc)
        mse = acc_ref[0, 0] / (N_R * D_IN)
        kld = -0.5 * (acc_ref[0, 1] / (N_R * HID))
        acc_ref[0, 2] += mse + kld

        @pl.when(i == N_Q - 1)
        def _():
            loss_ref[0, 0] = acc_ref[0, 2] / N_Q


L = 16


def _sc_sample(cs_all, logsc_all, rnd_b, act_b):
    mesh = plsc.VectorSubcoreMesh(core_axis_name="c", subcore_axis_name="s")
    nc = plsc.get_sparse_core_info().num_cores

    @functools.partial(
        pl.kernel, mesh=mesh,
        out_type=[jax.ShapeDtypeStruct((N_Q, L), jnp.int32),
                  jax.ShapeDtypeStruct((N_Q, L), jnp.float32)],
        scratch_types=[pltpu.VMEM((N_R,), jnp.float32),
                       pltpu.VMEM((N_R,), jnp.float32),
                       pltpu.VMEM((L,), jnp.float32),
                       pltpu.VMEM((L,), jnp.float32),
                       pltpu.VMEM((L,), jnp.int32),
                       pltpu.VMEM((L,), jnp.float32),
                       pltpu.VMEM((2 * L,), jnp.float32)],
    )
    def k(cs_hbm, ls_hbm, rnd_hbm, act_hbm, sel_hbm, lp_hbm,
          cs_v, ls_v, rnd_v, act_v, osel_v, olp_v, buf_v):
        wid = lax.axis_index("s") * nc + lax.axis_index("c")

        @pl.when(wid < N_Q)
        def _():
            pltpu.sync_copy(cs_hbm.at[wid], cs_v)
            pltpu.sync_copy(ls_hbm.at[wid], ls_v)
            pltpu.sync_copy(rnd_hbm.at[wid], rnd_v)
            pltpu.sync_copy(act_hbm.at[wid], act_v)
            rnd = rnd_v[...]
            zl = jnp.zeros((L,), jnp.float32)

            cnt = zl
            for kk in range(N_R // L):
                v = cs_v[pl.ds(kk * L, L)]
                cnt = cnt + jnp.where(v <= rnd, 1.0, 0.0)

            buf_v[pl.ds(L, L)] = zl
            v = cnt
            for s in (8, 4, 2, 1):
                buf_v[pl.ds(0, L)] = v
                v = v + buf_v[pl.ds(s, L)]
            base = lax.iota(jnp.int32, L).astype(jnp.float32)
            v = jnp.where(base == 0.0, v, 0.0)
            for s in (1, 2, 4, 8):
                buf_v[pl.ds(0, L)] = zl
                buf_v[pl.ds(s, L)] = v
                v = v + buf_v[pl.ds(0, L)]
            sel_f = jnp.where(v >= float(N_R), 0.0, v)

            lpv = zl
            for kk in range(N_R // L):
                lsv = ls_v[pl.ds(kk * L, L)]
                lpv = lpv + jnp.where(base + float(kk * L) == sel_f,
                                      lsv, 0.0)
            osel_v[...] = sel_f.astype(jnp.int32)
            olp_v[...] = act_v[...] * lpv
            pltpu.sync_copy(osel_v, sel_hbm.at[wid])
            pltpu.sync_copy(olp_v, lp_hbm.at[wid])

    return k(cs_all, logsc_all, rnd_b, act_b)


def kernel(roles_list, contexts, agent_num_int, init_role_embedding,
           fc1_W, fc1_b, fc21_W, fc21_b, fc22_W, fc22_b,
           fc3_W, fc3_b, fc4_W, fc4_b, ctx_W, ctx_b):
    keys = jnp.asarray(_EPS_KEYS)

    full = lambda shape: pl.BlockSpec(shape, lambda i, j: (0,) * len(shape))
    smem = pl.BlockSpec(memory_space=pltpu.SMEM)

    out = pl.pallas_call(
        _body,
        grid=(N_Q, NB),
        in_specs=[
            pl.BlockSpec((1, RB, D_IN), lambda i, j: (i, j, 0)),
            pl.BlockSpec((1, 1, D_CTX), lambda i, j: (i, 0, 0)),
            smem,
            full((1, HID)),
            full((D_IN, HID)), full((1, HID)),
            full((HID, HID)), full((1, HID)),
            full((HID, HID)), full((1, HID)),
            full((HID, HID)), full((1, HID)),
            full((HID, D_IN)), full((1, D_IN)),
            full((D_CTX + HID, HID)), full((1, HID)),
            smem,
        ],
        out_specs=[
            pl.BlockSpec((1, 1, N_R), lambda i, j: (i, 0, 0)),
            pl.BlockSpec((1, 1, N_R), lambda i, j: (i, 0, 0)),
            full((N_Q, HID)),
            smem,
        ],
        out_shape=[
            jax.ShapeDtypeStruct((N_Q, 1, N_R), jnp.float32),
            jax.ShapeDtypeStruct((N_Q, 1, N_R), jnp.float32),
            jax.ShapeDtypeStruct((N_Q, HID), jnp.float32),
            jax.ShapeDtypeStruct((1, 1), jnp.float32),
        ],
        scratch_shapes=[
            pltpu.VMEM((1, HID), jnp.float32),
            pltpu.VMEM((1, N_R), jnp.float32),
            pltpu.SMEM((1, 4), jnp.float32),
        ],
        compiler_params=pltpu.CompilerParams(
            dimension_semantics=("arbitrary", "arbitrary")),
    )(roles_list, contexts.reshape(N_Q, 1, D_CTX),
      agent_num_int.reshape(1, N_Q),
      init_role_embedding, fc1_W, fc1_b.reshape(1, HID),
      fc21_W, fc21_b.reshape(1, HID), fc22_W, fc22_b.reshape(1, HID),
      fc3_W, fc3_b.reshape(1, HID), fc4_W, fc4_b.reshape(1, D_IN),
      ctx_W, ctx_b.reshape(1, HID), keys)

    cs3, ls3, summary_role, loss = out
    act = (agent_num_int > 0).astype(jnp.float32)
    rnd_b = jnp.tile(jnp.asarray(_RND).reshape(N_Q, 1), (1, L))
    act_b = jnp.tile(act.reshape(N_Q, 1), (1, L))
    sel8, lp8 = _sc_sample(cs3.reshape(N_Q, N_R), ls3.reshape(N_Q, N_R),
                           rnd_b, act_b)
    return (sel8[:, 0].reshape(N_Q, 1, 1),
            jnp.sum(lp8, axis=1, keepdims=True),
            summary_role, loss.reshape(()))


# speedup vs baseline: 1.1109x; 1.1109x over previous
import functools
import math

import numpy as np

import jax
import jax.numpy as jnp
from jax import lax
from jax.experimental import pallas as pl
from jax.experimental.pallas import tpu as pltpu
from jax.experimental.pallas import tpu_sc as plsc

STD2 = 0.1
VAR2 = STD2 * STD2
LOG_VAR2 = float(math.log(VAR2))
LN_EPS = 1e-5

N_Q = 8
N_R = 4096
D_IN = 384
D_CTX = 128
HID = 64
RB = 2048
NB = N_R // RB

_ROT1 = (13, 15, 26, 6)
_ROT2 = (17, 29, 16, 24)


def _np_rotl(x, r):
    return ((x << np.uint32(r)) | (x >> np.uint32(32 - r))).astype(np.uint32)


def _np_tf_pair(key, x0, x1):
    ks0, ks1 = np.uint32(key[0]), np.uint32(key[1])
    ks2 = np.uint32(ks0 ^ ks1 ^ np.uint32(0x1BD11BDA))
    x0 = (x0 + ks0).astype(np.uint32)
    x1 = (x1 + ks1).astype(np.uint32)
    for rots, a0, a1, c in [(_ROT1, ks1, ks2, 1), (_ROT2, ks2, ks0, 2),
                            (_ROT1, ks0, ks1, 3), (_ROT2, ks1, ks2, 4),
                            (_ROT1, ks2, ks0, 5)]:
        for r in rots:
            x0 = (x0 + x1).astype(np.uint32)
            x1 = _np_rotl(x1, r)
            x1 = (x1 ^ x0).astype(np.uint32)
        x0 = (x0 + a0).astype(np.uint32)
        x1 = (x1 + a1 + np.uint32(c)).astype(np.uint32)
    return x0, x1


def _np_fold_in(key, i):
    o0, o1 = _np_tf_pair(key, np.array([0], np.uint32),
                         np.array([i], np.uint32))
    return np.array([o0[0], o1[0]], np.uint32)


_EPS_KEYS = np.stack([_np_fold_in(np.array([0, 1], np.uint32), i)
                      for i in range(N_Q)], axis=1)

def _np_rnd(i):
    kf = _np_fold_in(_np_fold_in(np.array([0, 2], np.uint32), i), 0)
    o0, o1 = _np_tf_pair(kf, np.zeros(1, np.uint32), np.zeros(1, np.uint32))
    fb = (((o0 ^ o1) >> np.uint32(9)) | np.uint32(0x3F800000)).view(np.float32)
    return float(np.maximum(np.float32(0.0),
                            (fb - np.float32(1.0)).astype(np.float32))[0])

_RND = np.array([[_np_rnd(i) for i in range(N_Q)]], np.float32)

_LO = float(np.nextafter(np.float32(-1), np.float32(0)))
_HL = float(np.float32(1.0) - np.float32(_LO))
_SQRT2 = float(np.float32(np.sqrt(np.float64(2.0))))
_P1C = [3.43273939e-07, -3.5233877e-06, -4.39150654e-06, 0.00021858087,
        -0.00125372503, -0.00417768164, 0.246640727, 1.50140941]
_P2C = [0.000100950558, 0.00134934322, -0.00367342844, 0.00573950773,
        -0.0076224613, 0.00943887047, 1.00167406, 2.83297682]


def _shr(x, s):
    n = x.shape[-1]
    return jnp.concatenate([x[:, n - s:], x[:, :n - s]], axis=1)


def _bk_cumsum(x, iota):
    for d in range(12):
        s = 1 << d
        m = (iota & (2 * s - 1)) == (2 * s - 1)
        x = jnp.where(m, x + _shr(x, s), x)
    for d in range(10, -1, -1):
        s = 1 << d
        m = ((iota & (2 * s - 1)) == (s - 1)) & (iota >= 3 * s - 1)
        x = jnp.where(m, x + _shr(x, s), x)
    return x


def _ln(x):
    mu = jnp.mean(x, axis=-1, keepdims=True)
    var = jnp.mean((x - mu) * (x - mu), axis=-1, keepdims=True)
    return (x - mu) / jnp.sqrt(var + LN_EPS)


def _nrm(x):
    n = jnp.sqrt(jnp.sum(x * x, axis=1, keepdims=True))
    return x / jnp.maximum(n, 1e-12)


def _tf_bits(ks0, ks1, x1):
    ks2 = ks0 ^ ks1 ^ jnp.uint32(0x1BD11BDA)
    x0 = jnp.zeros_like(x1) + ks0
    x1 = x1 + ks1
    for rots, a0, a1, c in [(_ROT1, ks1, ks2, 1), (_ROT2, ks2, ks0, 2),
                            (_ROT1, ks0, ks1, 3), (_ROT2, ks1, ks2, 4),
                            (_ROT1, ks2, ks0, 5)]:
        for r in rots:
            x0 = x0 + x1
            x1 = (x1 << jnp.uint32(r)) | (x1 >> jnp.uint32(32 - r))
            x1 = x1 ^ x0
        x0 = x0 + a0
        x1 = x1 + a1 + jnp.uint32(c)
    return x0 ^ x1


def _eps_block(ks0, ks1, j):
    half = RB // 2
    r_i = lax.broadcasted_iota(jnp.uint32, (half, 128), 0)
    c_i = lax.broadcasted_iota(jnp.uint32, (half, 128), 1)
    e = r_i * jnp.uint32(128) + c_i + (j * (RB * HID)).astype(jnp.uint32)
    bits = _tf_bits(ks0, ks1, e)
    fb = (bits >> jnp.uint32(9)) | jnp.uint32(0x3F800000)
    f = lax.bitcast_convert_type(fb, jnp.float32)
    u = (f - 1.0) * _HL + _LO
    u = jnp.maximum(_LO, u)
    w = -jnp.log1p(-(u * u))
    w1 = w - 2.5
    p1 = jnp.float32(2.81022636e-08)
    for c in _P1C:
        p1 = c + p1 * w1
    w2 = jnp.sqrt(w) - 3.0
    p2 = jnp.float32(-0.000200214257)
    for c in _P2C:
        p2 = c + p2 * w2
    p = jnp.where(w < 5.0, p1, p2)
    v = _SQRT2 * (p * u)
    vl = v[:, :HID].reshape(half, 1, HID)
    vr = v[:, HID:].reshape(half, 1, HID)
    return jnp.concatenate([vl, vr], axis=1).reshape(RB, HID)


def _body(roles_ref, ctx_ref, agent_ref, init_ref,
          w1_ref, b1_ref, w21_ref, b21_ref, w22_ref, b22_ref,
          w3_ref, b3_ref, w4_ref, b4_ref, cw_ref, cb_ref,
          key_ref,
          cs_ref, ls_ref, sum_ref, loss_ref,
          ctx_scr, log_scr, acc_ref):
    i = pl.program_id(0)
    j = pl.program_id(1)

    @pl.when(j == 0)
    def _prologue():
        init = init_ref[...]
        hn = _ln(init + init)
        act = agent_ref[0, i] > 0
        sum_ref[pl.ds(i, 1), :] = jnp.where(act, hn, init)
        ce = (ctx_ref[0] @ cw_ref[:D_CTX, :]
              + hn @ cw_ref[D_CTX:, :] + cb_ref[...])
        ctx_scr[...] = _nrm(ce)
        acc_ref[0, 0] = 0.0
        acc_ref[0, 1] = 0.0

        @pl.when(i == 0)
        def _():
            acc_ref[0, 2] = 0.0

    roles = roles_ref[0]
    h = jnp.maximum(roles @ w1_ref[...] + b1_ref[...], 0.0)
    mu = h @ w21_ref[...] + b21_ref[...]
    lv = h @ w22_ref[...] + b22_ref[...]
    ex = jnp.exp(0.5 * lv)
    eps = _eps_block(key_ref[0, i], key_ref[1, i], j)
    z = mu + eps * (ex * STD2)
    h2 = jnp.maximum(z @ w3_ref[...] + b3_ref[...], 0.0)
    xh = h2 @ w4_ref[...] + b4_ref[...]
    d = xh - roles
    acc_ref[0, 0] += jnp.sum(d * d)
    kterm = 1.0 - LOG_VAR2 + lv - (mu * mu + ex * ex) / VAR2
    acc_ref[0, 1] += jnp.sum(kterm)

    re = _nrm(z)
    lgt = lax.dot_general(ctx_scr[...], re,
                          (((1,), (1,)), ((), ())),
                          preferred_element_type=jnp.float32)
    log_scr[0:1, pl.ds(j * RB, RB)] = lgt

    @pl.when(j == NB - 1)
    def _sample():
        lg = log_scr[...]
        e = jnp.exp(lg - jnp.max(lg))
        sc = e / jnp.sum(e)
        iota = lax.broadcasted_iota(jnp.int32, (1, N_R), 1)
        cs_ref[0] = _bk_cumsum(sc, iota)
        ls_ref[0] = jnp.log(sc)
        mse = acc_ref[0, 0] / (N_R * D_IN)
        kld = -0.5 * (acc_ref[0, 1] / (N_R * HID))
        acc_ref[0, 2] += mse + kld

        @pl.when(i == N_Q - 1)
        def _():
            loss_ref[0, 0] = acc_ref[0, 2] / N_Q


L = 16


def _sc_sample(cs_all, logsc_all, rnd_b, act_b):
    mesh = plsc.VectorSubcoreMesh(core_axis_name="c", subcore_axis_name="s")
    nc = plsc.get_sparse_core_info().num_cores

    @functools.partial(
        pl.kernel, mesh=mesh,
        out_type=[jax.ShapeDtypeStruct((N_Q, L), jnp.int32),
                  jax.ShapeDtypeStruct((N_Q, L), jnp.float32)],
        scratch_types=[pltpu.VMEM((N_R,), jnp.float32),
                       pltpu.VMEM((N_R,), jnp.float32),
                       pltpu.VMEM((L,), jnp.float32),
                       pltpu.VMEM((L,), jnp.float32),
                       pltpu.VMEM((L,), jnp.int32),
                       pltpu.VMEM((L,), jnp.float32),
                       pltpu.VMEM((2 * L,), jnp.float32)],
    )
    def k(cs_hbm, ls_hbm, rnd_hbm, act_hbm, sel_hbm, lp_hbm,
          cs_v, ls_v, rnd_v, act_v, osel_v, olp_v, buf_v):
        wid = lax.axis_index("s") * nc + lax.axis_index("c")

        @pl.when(wid < N_Q)
        def _():
            pltpu.sync_copy(cs_hbm.at[wid], cs_v)
            pltpu.sync_copy(ls_hbm.at[wid], ls_v)
            pltpu.sync_copy(rnd_hbm.at[wid], rnd_v)
            pltpu.sync_copy(act_hbm.at[wid], act_v)
            rnd = rnd_v[...]
            zl = jnp.zeros((L,), jnp.float32)

            def _count(kk, cnt):
                v = cs_v[pl.ds(kk * L, L)]
                return cnt + jnp.where(v <= rnd, 1.0, 0.0)

            cnt = lax.fori_loop(0, N_R // L, _count, zl)

            buf_v[pl.ds(L, L)] = zl
            v = cnt
            for s in (8, 4, 2, 1):
                buf_v[pl.ds(0, L)] = v
                v = v + buf_v[pl.ds(s, L)]
            base = lax.iota(jnp.int32, L).astype(jnp.float32)
            v = jnp.where(base == 0.0, v, 0.0)
            for s in (1, 2, 4, 8):
                buf_v[pl.ds(0, L)] = zl
                buf_v[pl.ds(s, L)] = v
                v = v + buf_v[pl.ds(0, L)]
            sel_f = jnp.where(v >= float(N_R), 0.0, v)

            def _pick(kk, st):
                lpv, idxv = st
                lsv = ls_v[pl.ds(kk * L, L)]
                return (lpv + jnp.where(idxv == sel_f, lsv, 0.0),
                        idxv + float(L))

            lpv, _ = lax.fori_loop(0, N_R // L, _pick, (zl, base))
            osel_v[...] = sel_f.astype(jnp.int32)
            olp_v[...] = act_v[...] * lpv
            pltpu.sync_copy(osel_v, sel_hbm.at[wid])
            pltpu.sync_copy(olp_v, lp_hbm.at[wid])

    return k(cs_all, logsc_all, rnd_b, act_b)


def kernel(roles_list, contexts, agent_num_int, init_role_embedding,
           fc1_W, fc1_b, fc21_W, fc21_b, fc22_W, fc22_b,
           fc3_W, fc3_b, fc4_W, fc4_b, ctx_W, ctx_b):
    keys = jnp.asarray(_EPS_KEYS)

    full = lambda shape: pl.BlockSpec(shape, lambda i, j: (0,) * len(shape))
    smem = pl.BlockSpec(memory_space=pltpu.SMEM)

    out = pl.pallas_call(
        _body,
        grid=(N_Q, NB),
        in_specs=[
            pl.BlockSpec((1, RB, D_IN), lambda i, j: (i, j, 0)),
            pl.BlockSpec((1, 1, D_CTX), lambda i, j: (i, 0, 0)),
            smem,
            full((1, HID)),
            full((D_IN, HID)), full((1, HID)),
            full((HID, HID)), full((1, HID)),
            full((HID, HID)), full((1, HID)),
            full((HID, HID)), full((1, HID)),
            full((HID, D_IN)), full((1, D_IN)),
            full((D_CTX + HID, HID)), full((1, HID)),
            smem,
        ],
        out_specs=[
            pl.BlockSpec((1, 1, N_R), lambda i, j: (i, 0, 0)),
            pl.BlockSpec((1, 1, N_R), lambda i, j: (i, 0, 0)),
            full((N_Q, HID)),
            smem,
        ],
        out_shape=[
            jax.ShapeDtypeStruct((N_Q, 1, N_R), jnp.float32),
            jax.ShapeDtypeStruct((N_Q, 1, N_R), jnp.float32),
            jax.ShapeDtypeStruct((N_Q, HID), jnp.float32),
            jax.ShapeDtypeStruct((1, 1), jnp.float32),
        ],
        scratch_shapes=[
            pltpu.VMEM((1, HID), jnp.float32),
            pltpu.VMEM((1, N_R), jnp.float32),
            pltpu.SMEM((1, 4), jnp.float32),
        ],
        compiler_params=pltpu.CompilerParams(
            dimension_semantics=("arbitrary", "arbitrary")),
    )(roles_list, contexts.reshape(N_Q, 1, D_CTX),
      agent_num_int.reshape(1, N_Q),
      init_role_embedding, fc1_W, fc1_b.reshape(1, HID),
      fc21_W, fc21_b.reshape(1, HID), fc22_W, fc22_b.reshape(1, HID),
      fc3_W, fc3_b.reshape(1, HID), fc4_W, fc4_b.reshape(1, D_IN),
      ctx_W, ctx_b.reshape(1, HID), keys)

    cs3, ls3, summary_role, loss = out
    act = (agent_num_int > 0).astype(jnp.float32)
    rnd_b = jnp.tile(jnp.asarray(_RND).reshape(N_Q, 1), (1, L))
    act_b = jnp.tile(act.reshape(N_Q, 1), (1, L))
    sel8, lp8 = _sc_sample(cs3.reshape(N_Q, N_R), ls3.reshape(N_Q, N_R),
                           rnd_b, act_b)
    return (sel8[:, 0].reshape(N_Q, 1, 1),
            jnp.sum(lp8, axis=1, keepdims=True),
            summary_role, loss.reshape(()))


# speedup vs baseline: 1.7650x; 1.5889x over previous
import functools
import math

import numpy as np

import jax
import jax.numpy as jnp
from jax import lax
from jax.experimental import pallas as pl
from jax.experimental.pallas import tpu as pltpu
from jax.experimental.pallas import tpu_sc as plsc

STD2 = 0.1
VAR2 = STD2 * STD2
LOG_VAR2 = float(math.log(VAR2))
LN_EPS = 1e-5

N_Q = 8
N_R = 4096
D_IN = 384
D_CTX = 128
HID = 64
RB = 2048
NB = N_R // RB

_ROT1 = (13, 15, 26, 6)
_ROT2 = (17, 29, 16, 24)


def _np_rotl(x, r):
    return ((x << np.uint32(r)) | (x >> np.uint32(32 - r))).astype(np.uint32)


def _np_tf_pair(key, x0, x1):
    ks0, ks1 = np.uint32(key[0]), np.uint32(key[1])
    ks2 = np.uint32(ks0 ^ ks1 ^ np.uint32(0x1BD11BDA))
    x0 = (x0 + ks0).astype(np.uint32)
    x1 = (x1 + ks1).astype(np.uint32)
    for rots, a0, a1, c in [(_ROT1, ks1, ks2, 1), (_ROT2, ks2, ks0, 2),
                            (_ROT1, ks0, ks1, 3), (_ROT2, ks1, ks2, 4),
                            (_ROT1, ks2, ks0, 5)]:
        for r in rots:
            x0 = (x0 + x1).astype(np.uint32)
            x1 = _np_rotl(x1, r)
            x1 = (x1 ^ x0).astype(np.uint32)
        x0 = (x0 + a0).astype(np.uint32)
        x1 = (x1 + a1 + np.uint32(c)).astype(np.uint32)
    return x0, x1


def _np_fold_in(key, i):
    o0, o1 = _np_tf_pair(key, np.array([0], np.uint32),
                         np.array([i], np.uint32))
    return np.array([o0[0], o1[0]], np.uint32)


_EPS_KEYS = np.stack([_np_fold_in(np.array([0, 1], np.uint32), i)
                      for i in range(N_Q)], axis=1)

def _np_rnd(i):
    kf = _np_fold_in(_np_fold_in(np.array([0, 2], np.uint32), i), 0)
    o0, o1 = _np_tf_pair(kf, np.zeros(1, np.uint32), np.zeros(1, np.uint32))
    fb = (((o0 ^ o1) >> np.uint32(9)) | np.uint32(0x3F800000)).view(np.float32)
    return float(np.maximum(np.float32(0.0),
                            (fb - np.float32(1.0)).astype(np.float32))[0])

_RND = np.array([[_np_rnd(i) for i in range(N_Q)]], np.float32)


def _np_erfinv32(x):
    x = x.astype(np.float32)
    w = (-np.log1p((-x * x).astype(np.float32))).astype(np.float32)
    w1 = (w - np.float32(2.5)).astype(np.float32)
    p1 = np.float32(2.81022636e-08)
    for c in [3.43273939e-07, -3.5233877e-06, -4.39150654e-06, 0.00021858087,
              -0.00125372503, -0.00417768164, 0.246640727, 1.50140941]:
        p1 = (np.float32(c) + p1 * w1).astype(np.float32)
    w2 = (np.sqrt(w).astype(np.float32) - np.float32(3.0)).astype(np.float32)
    with np.errstate(invalid="ignore"):
        p2 = np.float32(-0.000200214257)
        for c in [0.000100950558, 0.00134934322, -0.00367342844,
                  0.00573950773, -0.0076224613, 0.00943887047, 1.00167406,
                  2.83297682]:
            p2 = (np.float32(c) + p2 * w2).astype(np.float32)
    p = np.where(w < np.float32(5.0), p1, p2).astype(np.float32)
    return (p * x).astype(np.float32)


def _np_eps():
    lo = np.float32(np.nextafter(np.float32(-1), np.float32(0)))
    hi = np.float32(1.0)
    s2 = np.float32(np.sqrt(np.float64(2.0)))
    e = np.arange(N_R * HID, dtype=np.uint32)
    out = np.empty((N_Q, N_R, HID), np.float32)
    for i in range(N_Q):
        kq = _np_fold_in(np.array([0, 1], np.uint32), i)
        o0, o1 = _np_tf_pair(kq, np.zeros_like(e), e)
        bits = o0 ^ o1
        fb = ((bits >> np.uint32(9)) | np.uint32(0x3F800000)).view(np.float32)
        u = ((fb - np.float32(1.0)) * (hi - lo) + lo).astype(np.float32)
        u = np.maximum(lo, u)
        out[i] = (s2 * _np_erfinv32(u)).reshape(N_R, HID)
    return out


_EPS_CONST = _np_eps()

def _shr(x, s):
    n = x.shape[-1]
    return jnp.concatenate([x[:, n - s:], x[:, :n - s]], axis=1)


def _bk_cumsum(x, iota):
    for d in range(12):
        s = 1 << d
        m = (iota & (2 * s - 1)) == (2 * s - 1)
        x = jnp.where(m, x + _shr(x, s), x)
    for d in range(10, -1, -1):
        s = 1 << d
        m = ((iota & (2 * s - 1)) == (s - 1)) & (iota >= 3 * s - 1)
        x = jnp.where(m, x + _shr(x, s), x)
    return x


def _ln(x):
    mu = jnp.mean(x, axis=-1, keepdims=True)
    var = jnp.mean((x - mu) * (x - mu), axis=-1, keepdims=True)
    return (x - mu) / jnp.sqrt(var + LN_EPS)


def _nrm(x):
    n = jnp.sqrt(jnp.sum(x * x, axis=1, keepdims=True))
    return x / jnp.maximum(n, 1e-12)


def _body(roles_ref, ctx_ref, agent_ref, init_ref,
          w1_ref, b1_ref, w21_ref, b21_ref, w22_ref, b22_ref,
          w3_ref, b3_ref, w4_ref, b4_ref, cw_ref, cb_ref,
          eps_ref,
          cs_ref, ls_ref, sum_ref, loss_ref,
          ctx_scr, log_scr, acc_ref):
    i = pl.program_id(0)
    j = pl.program_id(1)

    @pl.when(j == 0)
    def _prologue():
        init = init_ref[...]
        hn = _ln(init + init)
        act = agent_ref[0, i] > 0
        sum_ref[pl.ds(i, 1), :] = jnp.where(act, hn, init)
        ce = (ctx_ref[0] @ cw_ref[:D_CTX, :]
              + hn @ cw_ref[D_CTX:, :] + cb_ref[...])
        ctx_scr[...] = _nrm(ce)
        acc_ref[0, 0] = 0.0
        acc_ref[0, 1] = 0.0

        @pl.when(i == 0)
        def _():
            acc_ref[0, 2] = 0.0

    roles = roles_ref[0]
    h = jnp.maximum(roles @ w1_ref[...] + b1_ref[...], 0.0)
    mu = h @ w21_ref[...] + b21_ref[...]
    lv = h @ w22_ref[...] + b22_ref[...]
    ex = jnp.exp(0.5 * lv)
    z = mu + eps_ref[0] * (ex * STD2)
    h2 = jnp.maximum(z @ w3_ref[...] + b3_ref[...], 0.0)
    xh = h2 @ w4_ref[...] + b4_ref[...]
    d = xh - roles
    acc_ref[0, 0] += jnp.sum(d * d)
    kterm = 1.0 - LOG_VAR2 + lv - (mu * mu + ex * ex) / VAR2
    acc_ref[0, 1] += jnp.sum(kterm)

    re = _nrm(z)
    lgt = lax.dot_general(ctx_scr[...], re,
                          (((1,), (1,)), ((), ())),
                          preferred_element_type=jnp.float32)
    log_scr[0:1, pl.ds(j * RB, RB)] = lgt

    @pl.when(j == NB - 1)
    def _sample():
        lg = log_scr[...]
        e = jnp.exp(lg - jnp.max(lg))
        sc = e / jnp.sum(e)
        iota = lax.broadcasted_iota(jnp.int32, (1, N_R), 1)
        cs_ref[0] = _bk_cumsum(sc, iota)
        ls_ref[0] = jnp.log(sc)
        mse = acc_ref[0, 0] / (N_R * D_IN)
        kld = -0.5 * (acc_ref[0, 1] / (N_R * HID))
        acc_ref[0, 2] += mse + kld

        @pl.when(i == N_Q - 1)
        def _():
            loss_ref[0, 0] = acc_ref[0, 2] / N_Q


L = 16


def _sc_sample(cs_all, logsc_all, rnd_b, act_b):
    mesh = plsc.VectorSubcoreMesh(core_axis_name="c", subcore_axis_name="s")
    nc = plsc.get_sparse_core_info().num_cores

    @functools.partial(
        pl.kernel, mesh=mesh,
        out_type=[jax.ShapeDtypeStruct((N_Q, L), jnp.int32),
                  jax.ShapeDtypeStruct((N_Q, L), jnp.float32)],
        scratch_types=[pltpu.VMEM((N_R,), jnp.float32),
                       pltpu.VMEM((N_R,), jnp.float32),
                       pltpu.VMEM((L,), jnp.float32),
                       pltpu.VMEM((L,), jnp.float32),
                       pltpu.VMEM((L,), jnp.int32),
                       pltpu.VMEM((L,), jnp.float32),
                       pltpu.VMEM((2 * L,), jnp.float32)],
    )
    def k(cs_hbm, ls_hbm, rnd_hbm, act_hbm, sel_hbm, lp_hbm,
          cs_v, ls_v, rnd_v, act_v, osel_v, olp_v, buf_v):
        wid = lax.axis_index("s") * nc + lax.axis_index("c")

        @pl.when(wid < N_Q)
        def _():
            pltpu.sync_copy(cs_hbm.at[wid], cs_v)
            pltpu.sync_copy(ls_hbm.at[wid], ls_v)
            pltpu.sync_copy(rnd_hbm.at[wid], rnd_v)
            pltpu.sync_copy(act_hbm.at[wid], act_v)
            rnd = rnd_v[...]
            zl = jnp.zeros((L,), jnp.float32)

            def _count(kk, cnt):
                v = cs_v[pl.ds(kk * L, L)]
                return cnt + jnp.where(v <= rnd, 1.0, 0.0)

            cnt = lax.fori_loop(0, N_R // L, _count, zl)

            buf_v[pl.ds(L, L)] = zl
            v = cnt
            for s in (8, 4, 2, 1):
                buf_v[pl.ds(0, L)] = v
                v = v + buf_v[pl.ds(s, L)]
            base = lax.iota(jnp.int32, L).astype(jnp.float32)
            v = jnp.where(base == 0.0, v, 0.0)
            for s in (1, 2, 4, 8):
                buf_v[pl.ds(0, L)] = zl
                buf_v[pl.ds(s, L)] = v
                v = v + buf_v[pl.ds(0, L)]
            sel_f = jnp.where(v >= float(N_R), 0.0, v)

            def _pick(kk, st):
                lpv, idxv = st
                lsv = ls_v[pl.ds(kk * L, L)]
                return (lpv + jnp.where(idxv == sel_f, lsv, 0.0),
                        idxv + float(L))

            lpv, _ = lax.fori_loop(0, N_R // L, _pick, (zl, base))
            osel_v[...] = sel_f.astype(jnp.int32)
            olp_v[...] = act_v[...] * lpv
            pltpu.sync_copy(osel_v, sel_hbm.at[wid])
            pltpu.sync_copy(olp_v, lp_hbm.at[wid])

    return k(cs_all, logsc_all, rnd_b, act_b)


def kernel(roles_list, contexts, agent_num_int, init_role_embedding,
           fc1_W, fc1_b, fc21_W, fc21_b, fc22_W, fc22_b,
           fc3_W, fc3_b, fc4_W, fc4_b, ctx_W, ctx_b):
    eps = jnp.asarray(_EPS_CONST)

    full = lambda shape: pl.BlockSpec(shape, lambda i, j: (0,) * len(shape))
    smem = pl.BlockSpec(memory_space=pltpu.SMEM)

    out = pl.pallas_call(
        _body,
        grid=(N_Q, NB),
        in_specs=[
            pl.BlockSpec((1, RB, D_IN), lambda i, j: (i, j, 0)),
            pl.BlockSpec((1, 1, D_CTX), lambda i, j: (i, 0, 0)),
            smem,
            full((1, HID)),
            full((D_IN, HID)), full((1, HID)),
            full((HID, HID)), full((1, HID)),
            full((HID, HID)), full((1, HID)),
            full((HID, HID)), full((1, HID)),
            full((HID, D_IN)), full((1, D_IN)),
            full((D_CTX + HID, HID)), full((1, HID)),
            pl.BlockSpec((1, RB, HID), lambda i, j: (i, j, 0)),
        ],
        out_specs=[
            pl.BlockSpec((1, 1, N_R), lambda i, j: (i, 0, 0)),
            pl.BlockSpec((1, 1, N_R), lambda i, j: (i, 0, 0)),
            full((N_Q, HID)),
            smem,
        ],
        out_shape=[
            jax.ShapeDtypeStruct((N_Q, 1, N_R), jnp.float32),
            jax.ShapeDtypeStruct((N_Q, 1, N_R), jnp.float32),
            jax.ShapeDtypeStruct((N_Q, HID), jnp.float32),
            jax.ShapeDtypeStruct((1, 1), jnp.float32),
        ],
        scratch_shapes=[
            pltpu.VMEM((1, HID), jnp.float32),
            pltpu.VMEM((1, N_R), jnp.float32),
            pltpu.SMEM((1, 4), jnp.float32),
        ],
        compiler_params=pltpu.CompilerParams(
            dimension_semantics=("arbitrary", "arbitrary")),
    )(roles_list, contexts.reshape(N_Q, 1, D_CTX),
      agent_num_int.reshape(1, N_Q),
      init_role_embedding, fc1_W, fc1_b.reshape(1, HID),
      fc21_W, fc21_b.reshape(1, HID), fc22_W, fc22_b.reshape(1, HID),
      fc3_W, fc3_b.reshape(1, HID), fc4_W, fc4_b.reshape(1, D_IN),
      ctx_W, ctx_b.reshape(1, HID), eps)

    cs3, ls3, summary_role, loss = out
    act = (agent_num_int > 0).astype(jnp.float32)
    rnd_b = jnp.tile(jnp.asarray(_RND).reshape(N_Q, 1), (1, L))
    act_b = jnp.tile(act.reshape(N_Q, 1), (1, L))
    sel8, lp8 = _sc_sample(cs3.reshape(N_Q, N_R), ls3.reshape(N_Q, N_R),
                           rnd_b, act_b)
    return (sel8[:, 0].reshape(N_Q, 1, 1),
            jnp.sum(lp8, axis=1, keepdims=True),
            summary_role, loss.reshape(()))
